# baseline (speedup 1.0000x reference)
import functools

import jax
import jax.numpy as jnp
from jax import lax
from jax.experimental import pallas as pl
from jax.experimental.pallas import tpu as pltpu
from jax.experimental.pallas import tpu_sc as plsc


def _gather_sc(emb_table, idx):
    B = idx.shape[0]
    V, D = emb_table.shape
    info = plsc.get_sparse_core_info()
    nw = info.num_cores * info.num_subcores
    b_per_w = B // nw
    mesh = plsc.VectorSubcoreMesh(core_axis_name="c", subcore_axis_name="s")

    @functools.partial(
        pl.kernel,
        mesh=mesh,
        compiler_params=pltpu.CompilerParams(use_tc_tiling_on_sc=False),
        out_type=jax.ShapeDtypeStruct((B, D), jnp.float32),
        scratch_types=[
            pltpu.VMEM((b_per_w,), jnp.int32),
            pltpu.VMEM((b_per_w, D), jnp.float32),
            pltpu.SemaphoreType.DMA,
        ],
    )
    def gather(table_hbm, idx_hbm, out_hbm, idx_v, rows_v, sem):
        wid = lax.axis_index("s") * info.num_cores + lax.axis_index("c")
        base = wid * b_per_w
        pltpu.sync_copy(idx_hbm.at[pl.ds(base, b_per_w)], idx_v)
        pltpu.async_copy(table_hbm.at[idx_v], rows_v, sem).wait()
        pltpu.sync_copy(rows_v, out_hbm.at[pl.ds(base, b_per_w)])

    return gather(emb_table, idx)


_BM = 32
_NBUF = 2


def _project_tc(embeds, Wh):
    B, D = embeds.shape
    V = Wh.shape[1]
    n_steps = B // _BM

    def body(e_ref, w_ref, o_hbm, obuf, sems):
        j = pl.program_id(0)
        slot = lax.rem(j, _NBUF)
        e_blk = e_ref[pl.ds(j * _BM, _BM), :].astype(jnp.bfloat16)

        for k in range(_NBUF):

            @pl.when(slot == k)
            def _(k=k):
                @pl.when(j >= _NBUF)
                def _():
                    pltpu.make_async_copy(
                        obuf.at[k], o_hbm.at[j - _NBUF], sems.at[k]
                    ).wait()

                obuf[k] = jnp.dot(
                    e_blk,
                    w_ref[...],
                    preferred_element_type=jnp.float32,
                )
                pltpu.make_async_copy(
                    obuf.at[k], o_hbm.at[j], sems.at[k]
                ).start()

        @pl.when(j == n_steps - 1)
        def _drain():
            for k in range(_NBUF):
                jc = n_steps - _NBUF + k
                pltpu.make_async_copy(
                    obuf.at[jc % _NBUF], o_hbm.at[jc], sems.at[jc % _NBUF]
                ).wait()

    grid_spec = pltpu.PrefetchScalarGridSpec(
        num_scalar_prefetch=0,
        grid=(n_steps,),
        in_specs=[
            pl.BlockSpec((B, D), lambda j: (0, 0)),
            pl.BlockSpec((D, V), lambda j: (0, 0)),
        ],
        out_specs=pl.BlockSpec(memory_space=pl.ANY),
        scratch_shapes=[
            pltpu.VMEM((_NBUF, _BM, V), jnp.float32),
            pltpu.SemaphoreType.DMA((_NBUF,)),
        ],
    )
    out3 = pl.pallas_call(
        body,
        grid_spec=grid_spec,
        out_shape=jax.ShapeDtypeStruct((n_steps, _BM, V), jnp.float32),
        compiler_params=pltpu.CompilerParams(
            dimension_semantics=("arbitrary",),
        ),
    )(embeds, Wh)
    return out3.reshape(B, V)


def kernel(inputs, emb_table, W, b):
    embeds = _gather_sc(emb_table, inputs.astype(jnp.int32))
    B = embeds.shape[0]
    e_aug = jnp.concatenate([embeds, jnp.ones((B, 1), jnp.float32)], axis=1)
    w_aug = jnp.concatenate([W, b[None, :]], axis=0).astype(jnp.bfloat16)
    return _project_tc(e_aug, w_aug)
